# baseline (speedup 1.0000x reference)
import functools

import jax
import jax.numpy as jnp
from jax import lax
from jax.experimental import pallas as pl
from jax.experimental.pallas import tpu as pltpu
from jax.experimental.pallas import tpu_sc as plsc

N = 10000
E = 320000
D = 128
NR = 8
NODE_EMB = 128

NC, NS = 2, 16
NW = NC * NS
CH = 128
NWE = 16
KCH = 160
G = 16
NG = KCH // G
EP = NWE * KCH * CH
NP = 10112
NP_T = NP // NS
CNT = 81920
CNT_T = CNT // NS

_SC_MESH = plsc.VectorSubcoreMesh(core_axis_name="c", subcore_axis_name="s")


_BC = 32000


def _prep_body(m_ref, src_ref, dst_ref, g_ref, k_ref):
    m = m_ref[...]
    mx = jnp.max(m, axis=0, keepdims=True)
    rid = lax.broadcasted_iota(jnp.int32, m.shape, 0)
    am = jnp.min(jnp.where(m == mx, rid, NR), axis=0, keepdims=True)
    g_ref[...] = am * N + src_ref[...]
    k_ref[...] = am * N + dst_ref[...]


def _prep(mT, src2, dst2):
    return pl.pallas_call(
        _prep_body,
        grid=(E // _BC,),
        in_specs=[pl.BlockSpec((NR, _BC), lambda i: (0, i)),
                  pl.BlockSpec((1, _BC), lambda i: (0, i)),
                  pl.BlockSpec((1, _BC), lambda i: (0, i))],
        out_specs=[pl.BlockSpec((1, _BC), lambda i: (0, i)),
                   pl.BlockSpec((1, _BC), lambda i: (0, i))],
        out_shape=[jax.ShapeDtypeStruct((1, E), jnp.int32),
                   jax.ShapeDtypeStruct((1, E), jnp.int32)],
    )(mT, src2, dst2)


_BN = 400


def _mm_body(x_ref, w_ref, y_ref):
    y_ref[0] = jnp.dot(x_ref[...], w_ref[0], preferred_element_type=jnp.float32)


def _mm(xin, ws):
    return pl.pallas_call(
        _mm_body,
        grid=(NR + 1, N // _BN),
        in_specs=[pl.BlockSpec((_BN, D), lambda r, n: (n, 0)),
                  pl.BlockSpec((1, D, D), lambda r, n: (r, 0, 0))],
        out_specs=pl.BlockSpec((1, _BN, D), lambda r, n: (r, n, 0)),
        out_shape=jax.ShapeDtypeStruct((NR + 1, N, D), jnp.float32),
    )(xin, ws)


def _comb_body(p_ref, y_ref, b_ref, o_ref, *, relu):
    v = p_ref[0] + p_ref[1] + y_ref[0] + b_ref[...]
    o_ref[...] = jnp.maximum(v, 0.0) if relu else v


def _combine(parts, y, b, relu):
    return pl.pallas_call(
        functools.partial(_comb_body, relu=relu),
        grid=(N // _BN,),
        in_specs=[pl.BlockSpec((2, _BN, D), lambda n: (0, n, 0)),
                  pl.BlockSpec((1, _BN, D), lambda n: (NR, n, 0)),
                  pl.BlockSpec((1, D), lambda n: (0, 0))],
        out_specs=pl.BlockSpec((_BN, D), lambda n: (n, 0)),
        out_shape=jax.ShapeDtypeStruct((N, D), jnp.float32),
    )(parts, y, b)


@functools.partial(
    pl.kernel,
    mesh=_SC_MESH,
    out_type=jax.ShapeDtypeStruct((2 * CNT,), jnp.float32),
    scratch_types=[
        pltpu.VMEM((KCH, CH), jnp.int32),
        pltpu.VMEM((CH,), jnp.float32),
        pltpu.VMEM((CNT_T,), jnp.float32),
        pltpu.VMEM_SHARED((CNT,), jnp.float32),
    ],
)
def _count_kernel(skey3, out, skv, onesv, zbuf, cnt):
    c = lax.axis_index("c")
    s = lax.axis_index("s")
    w = c * NS + s

    def zb(i, carry):
        zbuf[pl.ds(i * 16, 16)] = jnp.zeros((16,), jnp.float32)
        return carry

    lax.fori_loop(0, CNT_T // 16, zb, 0)
    for k in range(CH // 16):
        onesv[pl.ds(k * 16, 16)] = jnp.ones((16,), jnp.float32)
    pltpu.sync_copy(zbuf, cnt.at[pl.ds(s * CNT_T, CNT_T)])
    plsc.subcore_barrier()

    @pl.when(c == 0)
    def _work():
        pltpu.sync_copy(skey3.at[s], skv)

        def step(j, carry):
            pltpu.sync_copy(onesv, cnt.at[skv.at[j]], add=True)
            return carry

        lax.fori_loop(0, KCH, step, 0)

    plsc.subcore_barrier()
    pltpu.sync_copy(cnt.at[pl.ds(s * CNT_T, CNT_T)],
                    out.at[pl.ds(c * CNT + s * CNT_T, CNT_T)])


@functools.partial(
    pl.kernel,
    mesh=_SC_MESH,
    out_type=jax.ShapeDtypeStruct((NWE, KCH, CH), jnp.float32),
    scratch_types=[
        pltpu.VMEM((KCH, CH), jnp.int32),
        pltpu.VMEM((KCH, CH), jnp.float32),
        pltpu.VMEM((KCH, CH), jnp.float32),
        pltpu.VMEM((KCH, CH), jnp.float32),
        pltpu.SemaphoreType.DMA,
    ],
)
def _scale_kernel(c0, c1, skey3, out, skv, g0b, g1b, sbuf, sem):
    c = lax.axis_index("c")
    s = lax.axis_index("s")
    w = c * NS + s

    @pl.when(c == 0)
    def _work():
        _scale_work(c0, c1, skey3, out, skv, g0b, g1b, sbuf, sem, s)


def _scale_work(c0, c1, skey3, out, skv, g0b, g1b, sbuf, sem, w):
    pltpu.sync_copy(skey3.at[w], skv)

    W = 8
    for j in range(W):
        pltpu.async_copy(c0.at[skv.at[j]], g0b.at[j], sem)
        pltpu.async_copy(c1.at[skv.at[j]], g1b.at[j], sem)

    def step(j, carry):
        pltpu.make_async_copy(c0.at[skv.at[j]], g0b.at[j], sem).wait()
        pltpu.make_async_copy(c1.at[skv.at[j]], g1b.at[j], sem).wait()

        @pl.when(j + W < KCH)
        def _fire():
            pltpu.async_copy(c0.at[skv.at[j + W]], g0b.at[j + W], sem)
            pltpu.async_copy(c1.at[skv.at[j + W]], g1b.at[j + W], sem)

        for k in range(CH // 16):
            a = g0b[j, pl.ds(k * 16, 16)] + g1b[j, pl.ds(k * 16, 16)]
            sbuf[j, pl.ds(k * 16, 16)] = 1.0 / jnp.maximum(a, 1.0)
        return carry

    lax.fori_loop(0, KCH, step, 0)
    pltpu.sync_copy(sbuf, out.at[w])


@functools.partial(
    pl.kernel,
    mesh=_SC_MESH,
    out_type=jax.ShapeDtypeStruct((2, NP, D), jnp.float32),
    scratch_types=[
        pltpu.VMEM((G, CH), jnp.int32),
        pltpu.VMEM((G, CH), jnp.int32),
        pltpu.VMEM((G, CH), jnp.float32),
        pltpu.VMEM((CH, D), jnp.float32),
        pltpu.VMEM((CH, D), jnp.float32),
        pltpu.VMEM_SHARED((NP, D), jnp.float32),
        pltpu.SemaphoreType.DMA,
        pltpu.SemaphoreType.DMA,
    ],
)
def _agg_kernel(yflat, gkey3, dst3, s3, out,
                gkv, dstv, sv, rowsA, rowsB, acc, semA, semB):
    c = lax.axis_index("c")
    s = lax.axis_index("s")
    w = c * NS + s

    def zr(e, carry):
        for k in range(D // 16):
            rowsA[e, pl.ds(k * 16, 16)] = jnp.zeros((16,), jnp.float32)
        return carry

    lax.fori_loop(0, CH, zr, 0)
    base = s * NP_T
    nfull = NP_T // CH
    rem = NP_T - nfull * CH
    for q in range(nfull):
        pltpu.sync_copy(rowsA, acc.at[pl.ds(base + q * CH, CH)])
    pltpu.sync_copy(rowsA.at[pl.ds(0, rem)],
                    acc.at[pl.ds(base + nfull * CH, rem)])
    plsc.subcore_barrier()

    @pl.when(c == 0)
    def _edges():
        _agg_edges(yflat, gkey3, dst3, s3, gkv, dstv, sv,
                   rowsA, rowsB, acc, semA, semB, s)

    plsc.subcore_barrier()
    pltpu.sync_copy(acc.at[pl.ds(base, NP_T)], out.at[c, pl.ds(base, NP_T)])


def _agg_edges(yflat, gkey3, dst3, s3, gkv, dstv, sv,
               rowsA, rowsB, acc, semA, semB, w):
    def _scale(rows, j):
        for b in range(CH // 16):
            sj = sv[j, pl.ds(b * 16, 16)]

            def sc16(l, c2, b=b, sj=sj):
                spl = sj.at[jnp.full((16,), l, jnp.int32)].get(
                    mode="promise_in_bounds")
                e = b * 16 + l
                for k in range(D // 16):
                    rows[e, pl.ds(k * 16, 16)] = (
                        rows[e, pl.ds(k * 16, 16)] * spl)
                return c2

            lax.fori_loop(0, 16, sc16, 0)

    def _process(rows, sem, j):
        pltpu.make_async_copy(yflat.at[gkv.at[j]], rows, sem).wait()
        _scale(rows, j)
        pltpu.sync_copy(rows, acc.at[dstv.at[j]], add=True)

        @pl.when(j + 2 < G)
        def _fire():
            pltpu.async_copy(yflat.at[gkv.at[j + 2]], rows, sem)

    def group(g, carry):
        pltpu.sync_copy(gkey3.at[w, pl.ds(g * G, G)], gkv)
        pltpu.sync_copy(dst3.at[w, pl.ds(g * G, G)], dstv)
        pltpu.sync_copy(s3.at[w, pl.ds(g * G, G)], sv)
        pltpu.async_copy(yflat.at[gkv.at[0]], rowsA, semA)
        pltpu.async_copy(yflat.at[gkv.at[1]], rowsB, semB)

        def pair(t, c2):
            _process(rowsA, semA, 2 * t)
            _process(rowsB, semB, 2 * t + 1)
            return c2

        lax.fori_loop(0, G // 2, pair, 0)
        return carry

    lax.fori_loop(0, NG, group, 0)


@jax.jit
def _run(x, edge_index, msg, W1, R1, b1, W2, R2, b2):
    src = edge_index[0]
    dst = edge_index[1]
    mT = jnp.transpose(msg[:, NODE_EMB:NODE_EMB + NR])
    gkey2, skey2 = _prep(mT, src.reshape(1, E), dst.reshape(1, E))
    pad = EP - E
    gkey3 = jnp.reshape(jnp.pad(jnp.reshape(gkey2, (E,)), (0, pad)),
                        (NWE, KCH, CH))
    skey3 = jnp.reshape(
        jnp.pad(jnp.reshape(skey2, (E,)), (0, pad), constant_values=NR * N),
        (NWE, KCH, CH))
    dst3 = jnp.reshape(jnp.pad(dst, (0, pad), constant_values=N),
                       (NWE, KCH, CH))

    cnt2 = _count_kernel(skey3)
    s3 = _scale_kernel(cnt2[:CNT], cnt2[CNT:], skey3)

    ws1 = jnp.concatenate([W1, R1[None]], axis=0)
    ws2 = jnp.concatenate([W2, R2[None]], axis=0)

    y1 = _mm(x, ws1)
    p1 = _agg_kernel(jnp.reshape(y1, ((NR + 1) * N, D)), gkey3, dst3, s3)
    h1 = _combine(p1, y1, jnp.reshape(b1, (1, D)), relu=True)

    y2 = _mm(h1, ws2)
    p2 = _agg_kernel(jnp.reshape(y2, ((NR + 1) * N, D)), gkey3, dst3, s3)
    return _combine(p2, y2, jnp.reshape(b2, (1, D)), relu=False)


def kernel(x, last_update, edge_index, t, msg, W1, R1, b1, W2, R2, b2):
    return _run(x, edge_index, msg, W1, R1, b1, W2, R2, b2)


# speedup vs baseline: 1.1288x; 1.1288x over previous
import functools

import jax
import jax.numpy as jnp
from jax import lax
from jax.experimental import pallas as pl
from jax.experimental.pallas import tpu as pltpu
from jax.experimental.pallas import tpu_sc as plsc

N = 10000
E = 320000
D = 128
NR = 8
NODE_EMB = 128

NC, NS = 2, 16
NW = NC * NS
CH = 128
KCH = 80
G = 16
NG = KCH // G
EP = NW * KCH * CH
NP = 10112
NP_T = NP // NS
CNT = 81920
CNT_T = CNT // NS

_SC_MESH = plsc.VectorSubcoreMesh(core_axis_name="c", subcore_axis_name="s")


_BC = 32000


def _prep_body(m_ref, src_ref, dst_ref, g_ref, k_ref):
    m = m_ref[...]
    mx = jnp.max(m, axis=0, keepdims=True)
    rid = lax.broadcasted_iota(jnp.int32, m.shape, 0)
    am = jnp.min(jnp.where(m == mx, rid, NR), axis=0, keepdims=True)
    g_ref[...] = am * N + src_ref[...]
    k_ref[...] = am * N + dst_ref[...]


def _prep(mT, src2, dst2):
    return pl.pallas_call(
        _prep_body,
        grid=(E // _BC,),
        in_specs=[pl.BlockSpec((NR, _BC), lambda i: (0, i)),
                  pl.BlockSpec((1, _BC), lambda i: (0, i)),
                  pl.BlockSpec((1, _BC), lambda i: (0, i))],
        out_specs=[pl.BlockSpec((1, _BC), lambda i: (0, i)),
                   pl.BlockSpec((1, _BC), lambda i: (0, i))],
        out_shape=[jax.ShapeDtypeStruct((1, E), jnp.int32),
                   jax.ShapeDtypeStruct((1, E), jnp.int32)],
    )(mT, src2, dst2)


_BN = 400


def _mm_body(x_ref, w_ref, y_ref):
    y_ref[0] = jnp.dot(x_ref[...], w_ref[0], preferred_element_type=jnp.float32)


def _mm(xin, ws):
    return pl.pallas_call(
        _mm_body,
        grid=(NR + 1, N // _BN),
        in_specs=[pl.BlockSpec((_BN, D), lambda r, n: (n, 0)),
                  pl.BlockSpec((1, D, D), lambda r, n: (r, 0, 0))],
        out_specs=pl.BlockSpec((1, _BN, D), lambda r, n: (r, n, 0)),
        out_shape=jax.ShapeDtypeStruct((NR + 1, N, D), jnp.float32),
    )(xin, ws)


def _comb_body(p_ref, y_ref, b_ref, o_ref, *, relu):
    v = p_ref[0] + p_ref[1] + y_ref[0] + b_ref[...]
    o_ref[...] = jnp.maximum(v, 0.0) if relu else v


def _combine(parts, y, b, relu):
    return pl.pallas_call(
        functools.partial(_comb_body, relu=relu),
        grid=(N // _BN,),
        in_specs=[pl.BlockSpec((2, _BN, D), lambda n: (0, n, 0)),
                  pl.BlockSpec((1, _BN, D), lambda n: (NR, n, 0)),
                  pl.BlockSpec((1, D), lambda n: (0, 0))],
        out_specs=pl.BlockSpec((_BN, D), lambda n: (n, 0)),
        out_shape=jax.ShapeDtypeStruct((N, D), jnp.float32),
    )(parts, y, b)


@functools.partial(
    pl.kernel,
    mesh=_SC_MESH,
    out_type=jax.ShapeDtypeStruct((2 * CNT,), jnp.float32),
    scratch_types=[
        pltpu.VMEM((KCH, CH), jnp.int32),
        pltpu.VMEM((CH,), jnp.float32),
        pltpu.VMEM((CNT_T,), jnp.float32),
        pltpu.VMEM_SHARED((CNT,), jnp.float32),
    ],
)
def _count_kernel(skey3, out, skv, onesv, zbuf, cnt):
    c = lax.axis_index("c")
    s = lax.axis_index("s")
    w = c * NS + s

    def zb(i, carry):
        zbuf[pl.ds(i * 16, 16)] = jnp.zeros((16,), jnp.float32)
        return carry

    lax.fori_loop(0, CNT_T // 16, zb, 0)
    for k in range(CH // 16):
        onesv[pl.ds(k * 16, 16)] = jnp.ones((16,), jnp.float32)
    pltpu.sync_copy(zbuf, cnt.at[pl.ds(s * CNT_T, CNT_T)])
    plsc.subcore_barrier()

    pltpu.sync_copy(skey3.at[w], skv)

    def step(j, carry):
        pltpu.sync_copy(onesv, cnt.at[skv.at[j]], add=True)
        return carry

    lax.fori_loop(0, KCH, step, 0)
    plsc.subcore_barrier()
    pltpu.sync_copy(cnt.at[pl.ds(s * CNT_T, CNT_T)],
                    out.at[pl.ds(c * CNT + s * CNT_T, CNT_T)])


def _zero_acc(rowsA, acc, s):
    def zr(e, carry):
        for k in range(D // 16):
            rowsA[e, pl.ds(k * 16, 16)] = jnp.zeros((16,), jnp.float32)
        return carry

    lax.fori_loop(0, CH, zr, 0)
    base = s * NP_T
    nfull = NP_T // CH
    rem = NP_T - nfull * CH
    for q in range(nfull):
        pltpu.sync_copy(rowsA, acc.at[pl.ds(base + q * CH, CH)])
    pltpu.sync_copy(rowsA.at[pl.ds(0, rem)],
                    acc.at[pl.ds(base + nfull * CH, rem)])


def _scale_rows(rows, sv, j):
    for b in range(CH // 16):
        sj = sv[j, pl.ds(b * 16, 16)]

        def sc16(l, c2, b=b, sj=sj):
            spl = sj.at[jnp.full((16,), l, jnp.int32)].get(
                mode="promise_in_bounds")
            e = b * 16 + l
            for k in range(D // 16):
                rows[e, pl.ds(k * 16, 16)] = (
                    rows[e, pl.ds(k * 16, 16)] * spl)
            return c2

        lax.fori_loop(0, 16, sc16, 0)


@functools.partial(
    pl.kernel,
    mesh=_SC_MESH,
    out_type=[jax.ShapeDtypeStruct((2, NP, D), jnp.float32),
              jax.ShapeDtypeStruct((NW, KCH, CH), jnp.float32)],
    scratch_types=[
        pltpu.VMEM((G, CH), jnp.int32),
        pltpu.VMEM((G, CH), jnp.int32),
        pltpu.VMEM((G, CH), jnp.int32),
        pltpu.VMEM((G, CH), jnp.float32),
        pltpu.VMEM((G, CH), jnp.float32),
        pltpu.VMEM((G, CH), jnp.float32),
        pltpu.VMEM((CH, D), jnp.float32),
        pltpu.VMEM((CH, D), jnp.float32),
        pltpu.VMEM_SHARED((NP, D), jnp.float32),
        pltpu.SemaphoreType.DMA,
        pltpu.SemaphoreType.DMA,
        pltpu.SemaphoreType.DMA,
    ],
)
def _agg1_kernel(yflat, gkey3, dst3, skey3, c0, c1, out, s3out,
                 gkv, dstv, skv, sv, c0b, c1b, rowsA, rowsB, acc,
                 semA, semB, semC):
    c = lax.axis_index("c")
    s = lax.axis_index("s")
    w = c * NS + s
    _zero_acc(rowsA, acc, s)
    plsc.subcore_barrier()

    def _fire_cnt(j):
        pltpu.async_copy(c0.at[skv.at[j]], c0b.at[j], semC)
        pltpu.async_copy(c1.at[skv.at[j]], c1b.at[j], semC)

    def _process(rows, sem, j):
        pltpu.make_async_copy(yflat.at[gkv.at[j]], rows, sem).wait()
        pltpu.make_async_copy(c0.at[skv.at[j]], c0b.at[j], semC).wait()
        pltpu.make_async_copy(c1.at[skv.at[j]], c1b.at[j], semC).wait()
        for k in range(CH // 16):
            a = c0b[j, pl.ds(k * 16, 16)] + c1b[j, pl.ds(k * 16, 16)]
            sv[j, pl.ds(k * 16, 16)] = 1.0 / jnp.maximum(a, 1.0)
        _scale_rows(rows, sv, j)
        pltpu.sync_copy(rows, acc.at[dstv.at[j]], add=True)

        @pl.when(j + 2 < G)
        def _fire():
            pltpu.async_copy(yflat.at[gkv.at[j + 2]], rows, sem)
            _fire_cnt(j + 2)

    def group(g, carry):
        pltpu.sync_copy(gkey3.at[w, pl.ds(g * G, G)], gkv)
        pltpu.sync_copy(dst3.at[w, pl.ds(g * G, G)], dstv)
        pltpu.sync_copy(skey3.at[w, pl.ds(g * G, G)], skv)
        pltpu.async_copy(yflat.at[gkv.at[0]], rowsA, semA)
        pltpu.async_copy(yflat.at[gkv.at[1]], rowsB, semB)
        _fire_cnt(0)
        _fire_cnt(1)

        def pair(t, c2):
            _process(rowsA, semA, 2 * t)
            _process(rowsB, semB, 2 * t + 1)
            return c2

        lax.fori_loop(0, G // 2, pair, 0)
        pltpu.sync_copy(sv, s3out.at[w, pl.ds(g * G, G)])
        return carry

    lax.fori_loop(0, NG, group, 0)
    plsc.subcore_barrier()
    base = s * NP_T
    pltpu.sync_copy(acc.at[pl.ds(base, NP_T)], out.at[c, pl.ds(base, NP_T)])


@functools.partial(
    pl.kernel,
    mesh=_SC_MESH,
    out_type=jax.ShapeDtypeStruct((2, NP, D), jnp.float32),
    scratch_types=[
        pltpu.VMEM((G, CH), jnp.int32),
        pltpu.VMEM((G, CH), jnp.int32),
        pltpu.VMEM((G, CH), jnp.float32),
        pltpu.VMEM((CH, D), jnp.float32),
        pltpu.VMEM((CH, D), jnp.float32),
        pltpu.VMEM_SHARED((NP, D), jnp.float32),
        pltpu.SemaphoreType.DMA,
        pltpu.SemaphoreType.DMA,
    ],
)
def _agg2_kernel(yflat, gkey3, dst3, s3, out,
                 gkv, dstv, sv, rowsA, rowsB, acc, semA, semB):
    c = lax.axis_index("c")
    s = lax.axis_index("s")
    w = c * NS + s
    _zero_acc(rowsA, acc, s)
    plsc.subcore_barrier()

    def _process(rows, sem, j):
        pltpu.make_async_copy(yflat.at[gkv.at[j]], rows, sem).wait()
        _scale_rows(rows, sv, j)
        pltpu.sync_copy(rows, acc.at[dstv.at[j]], add=True)

        @pl.when(j + 2 < G)
        def _fire():
            pltpu.async_copy(yflat.at[gkv.at[j + 2]], rows, sem)

    def group(g, carry):
        pltpu.sync_copy(gkey3.at[w, pl.ds(g * G, G)], gkv)
        pltpu.sync_copy(dst3.at[w, pl.ds(g * G, G)], dstv)
        pltpu.sync_copy(s3.at[w, pl.ds(g * G, G)], sv)
        pltpu.async_copy(yflat.at[gkv.at[0]], rowsA, semA)
        pltpu.async_copy(yflat.at[gkv.at[1]], rowsB, semB)

        def pair(t, c2):
            _process(rowsA, semA, 2 * t)
            _process(rowsB, semB, 2 * t + 1)
            return c2

        lax.fori_loop(0, G // 2, pair, 0)
        return carry

    lax.fori_loop(0, NG, group, 0)
    plsc.subcore_barrier()
    base = s * NP_T
    pltpu.sync_copy(acc.at[pl.ds(base, NP_T)], out.at[c, pl.ds(base, NP_T)])


@jax.jit
def _run(x, edge_index, msg, W1, R1, b1, W2, R2, b2):
    src = edge_index[0]
    dst = edge_index[1]
    mT = jnp.transpose(msg[:, NODE_EMB:NODE_EMB + NR])
    gkey2, skey2 = _prep(mT, src.reshape(1, E), dst.reshape(1, E))
    pad = EP - E
    gkey3 = jnp.reshape(jnp.pad(jnp.reshape(gkey2, (E,)), (0, pad)),
                        (NW, KCH, CH))
    skey3 = jnp.reshape(
        jnp.pad(jnp.reshape(skey2, (E,)), (0, pad), constant_values=NR * N),
        (NW, KCH, CH))
    dst3 = jnp.reshape(jnp.pad(dst, (0, pad), constant_values=N),
                       (NW, KCH, CH))

    cnt2 = _count_kernel(skey3)

    ws1 = jnp.concatenate([W1, R1[None]], axis=0)
    ws2 = jnp.concatenate([W2, R2[None]], axis=0)

    y1 = _mm(x, ws1)
    p1, s3 = _agg1_kernel(jnp.reshape(y1, ((NR + 1) * N, D)), gkey3, dst3,
                          skey3, cnt2[:CNT], cnt2[CNT:])
    h1 = _combine(p1, y1, jnp.reshape(b1, (1, D)), relu=True)

    y2 = _mm(h1, ws2)
    p2 = _agg2_kernel(jnp.reshape(y2, ((NR + 1) * N, D)), gkey3, dst3, s3)
    return _combine(p2, y2, jnp.reshape(b2, (1, D)), relu=False)


def kernel(x, last_update, edge_index, t, msg, W1, R1, b1, W2, R2, b2):
    return _run(x, edge_index, msg, W1, R1, b1, W2, R2, b2)


# speedup vs baseline: 1.3349x; 1.1826x over previous
import functools

import jax
import jax.numpy as jnp
from jax import lax
from jax.experimental import pallas as pl
from jax.experimental.pallas import tpu as pltpu
from jax.experimental.pallas import tpu_sc as plsc

N = 10000
E = 320000
D = 128
NR = 8
NODE_EMB = 128

NC, NS = 2, 16
NW = NC * NS
CH = 128
KCH = 79
EP = NW * KCH * CH
NP = 10112
NP_T = NP // NS
CNT = 81920
CNT_T = CNT // NS

_SC_MESH = plsc.VectorSubcoreMesh(core_axis_name="c", subcore_axis_name="s")


_BC = 32000


def _prep_body(m_ref, src_ref, dst_ref, g_ref, k_ref):
    m = m_ref[...]
    mx = jnp.max(m, axis=0, keepdims=True)
    rid = lax.broadcasted_iota(jnp.int32, m.shape, 0)
    am = jnp.min(jnp.where(m == mx, rid, NR), axis=0, keepdims=True)
    g_ref[...] = am * N + src_ref[...]
    k_ref[...] = am * N + dst_ref[...]


def _prep(mT, src2, dst2):
    return pl.pallas_call(
        _prep_body,
        grid=(E // _BC,),
        in_specs=[pl.BlockSpec((NR, _BC), lambda i: (0, i)),
                  pl.BlockSpec((1, _BC), lambda i: (0, i)),
                  pl.BlockSpec((1, _BC), lambda i: (0, i))],
        out_specs=[pl.BlockSpec((1, _BC), lambda i: (0, i)),
                   pl.BlockSpec((1, _BC), lambda i: (0, i))],
        out_shape=[jax.ShapeDtypeStruct((1, E), jnp.int32),
                   jax.ShapeDtypeStruct((1, E), jnp.int32)],
    )(mT, src2, dst2)


_BN = 400


def _mm_body(x_ref, w_ref, y_ref):
    y_ref[0] = jnp.dot(x_ref[...], w_ref[0], preferred_element_type=jnp.float32)


def _mm(xin, ws):
    return pl.pallas_call(
        _mm_body,
        grid=(NR + 1, N // _BN),
        in_specs=[pl.BlockSpec((_BN, D), lambda r, n: (n, 0)),
                  pl.BlockSpec((1, D, D), lambda r, n: (r, 0, 0))],
        out_specs=pl.BlockSpec((1, _BN, D), lambda r, n: (r, n, 0)),
        out_shape=jax.ShapeDtypeStruct((NR + 1, N, D), jnp.float32),
    )(xin, ws)


def _comb_body(p_ref, y_ref, b_ref, o_ref, *, relu):
    v = p_ref[0] + p_ref[1] + y_ref[0] + b_ref[...]
    o_ref[...] = jnp.maximum(v, 0.0) if relu else v


def _combine(parts, y, b, relu):
    return pl.pallas_call(
        functools.partial(_comb_body, relu=relu),
        grid=(N // _BN,),
        in_specs=[pl.BlockSpec((2, _BN, D), lambda n: (0, n, 0)),
                  pl.BlockSpec((1, _BN, D), lambda n: (NR, n, 0)),
                  pl.BlockSpec((1, D), lambda n: (0, 0))],
        out_specs=pl.BlockSpec((_BN, D), lambda n: (n, 0)),
        out_shape=jax.ShapeDtypeStruct((N, D), jnp.float32),
    )(parts, y, b)


@functools.partial(
    pl.kernel,
    mesh=_SC_MESH,
    out_type=jax.ShapeDtypeStruct((2 * CNT,), jnp.float32),
    scratch_types=[
        pltpu.VMEM((KCH, CH), jnp.int32),
        pltpu.VMEM((CH,), jnp.float32),
        pltpu.VMEM((CNT_T,), jnp.float32),
        pltpu.VMEM_SHARED((CNT,), jnp.float32),
    ],
)
def _count_kernel(skey3, out, skv, onesv, zbuf, cnt):
    c = lax.axis_index("c")
    s = lax.axis_index("s")
    w = c * NS + s

    def zb(i, carry):
        zbuf[pl.ds(i * 16, 16)] = jnp.zeros((16,), jnp.float32)
        return carry

    lax.fori_loop(0, CNT_T // 16, zb, 0)
    for k in range(CH // 16):
        onesv[pl.ds(k * 16, 16)] = jnp.ones((16,), jnp.float32)
    pltpu.sync_copy(zbuf, cnt.at[pl.ds(s * CNT_T, CNT_T)])
    plsc.subcore_barrier()

    pltpu.sync_copy(skey3.at[w], skv)

    def step(j, carry):
        pltpu.sync_copy(onesv, cnt.at[skv.at[j]], add=True)
        return carry

    lax.fori_loop(0, KCH, step, 0)
    plsc.subcore_barrier()
    pltpu.sync_copy(cnt.at[pl.ds(s * CNT_T, CNT_T)],
                    out.at[pl.ds(c * CNT + s * CNT_T, CNT_T)])


@functools.partial(
    pl.kernel,
    mesh=_SC_MESH,
    out_type=jax.ShapeDtypeStruct((NW, KCH, CH), jnp.float32),
    scratch_types=[
        pltpu.VMEM((KCH, CH), jnp.int32),
        pltpu.VMEM((KCH, CH), jnp.float32),
        pltpu.VMEM((KCH, CH), jnp.float32),
        pltpu.VMEM((KCH, CH), jnp.float32),
        pltpu.SemaphoreType.DMA,
    ],
)
def _scale_kernel(c0, c1, skey3, out, skv, g0b, g1b, sbuf, sem):
    c = lax.axis_index("c")
    s = lax.axis_index("s")
    w = c * NS + s
    pltpu.sync_copy(skey3.at[w], skv)

    W = 8
    for j in range(W):
        pltpu.async_copy(c0.at[skv.at[j]], g0b.at[j], sem)
        pltpu.async_copy(c1.at[skv.at[j]], g1b.at[j], sem)

    def step(j, carry):
        pltpu.make_async_copy(c0.at[skv.at[j]], g0b.at[j], sem).wait()
        pltpu.make_async_copy(c1.at[skv.at[j]], g1b.at[j], sem).wait()

        @pl.when(j + W < KCH)
        def _fire():
            pltpu.async_copy(c0.at[skv.at[j + W]], g0b.at[j + W], sem)
            pltpu.async_copy(c1.at[skv.at[j + W]], g1b.at[j + W], sem)

        for k in range(CH // 16):
            a = g0b[j, pl.ds(k * 16, 16)] + g1b[j, pl.ds(k * 16, 16)]
            sbuf[j, pl.ds(k * 16, 16)] = 1.0 / jnp.maximum(a, 1.0)
        return carry

    lax.fori_loop(0, KCH, step, 0)
    pltpu.sync_copy(sbuf, out.at[w])


@functools.partial(
    pl.kernel,
    mesh=_SC_MESH,
    out_type=jax.ShapeDtypeStruct((2, NP, D), jnp.float32),
    scratch_types=[
        pltpu.VMEM((KCH, CH), jnp.int32),
        pltpu.VMEM((KCH, CH), jnp.int32),
        pltpu.VMEM((KCH, CH), jnp.float32),
        pltpu.VMEM((CH, D), jnp.float32),
        pltpu.VMEM_SHARED((NP, D), jnp.float32),
        pltpu.SemaphoreType.DMA,
    ],
)
def _agg_kernel(yflat, gkey3, dst3, s3, out, gkv, dstv, sv, rows, acc, sem):
    c = lax.axis_index("c")
    s = lax.axis_index("s")
    w = c * NS + s

    def zr(e, carry):
        for k in range(D // 16):
            rows[e, pl.ds(k * 16, 16)] = jnp.zeros((16,), jnp.float32)
        return carry

    lax.fori_loop(0, CH, zr, 0)
    base = s * NP_T
    nfull = NP_T // CH
    rem = NP_T - nfull * CH
    for q in range(nfull):
        pltpu.sync_copy(rows, acc.at[pl.ds(base + q * CH, CH)])
    pltpu.sync_copy(rows.at[pl.ds(0, rem)],
                    acc.at[pl.ds(base + nfull * CH, rem)])
    plsc.subcore_barrier()

    pltpu.sync_copy(gkey3.at[w], gkv)
    pltpu.sync_copy(dst3.at[w], dstv)
    pltpu.sync_copy(s3.at[w], sv)

    def step(j, carry):
        pltpu.async_copy(yflat.at[gkv.at[j]], rows, sem).wait()

        for b in range(CH // 16):
            sj = sv[j, pl.ds(b * 16, 16)]

            def sc16(l, c2, b=b, sj=sj):
                spl = sj.at[jnp.full((16,), l, jnp.int32)].get(
                    mode="promise_in_bounds")
                e = b * 16 + l
                for k in range(D // 16):
                    rows[e, pl.ds(k * 16, 16)] = (
                        rows[e, pl.ds(k * 16, 16)] * spl)
                return c2

            lax.fori_loop(0, 16, sc16, 0)
        pltpu.sync_copy(rows, acc.at[dstv.at[j]], add=True)
        return carry

    lax.fori_loop(0, KCH, step, 0)
    plsc.subcore_barrier()
    pltpu.sync_copy(acc.at[pl.ds(base, NP_T)], out.at[c, pl.ds(base, NP_T)])


@jax.jit
def _run(x, edge_index, msg, W1, R1, b1, W2, R2, b2):
    src = edge_index[0]
    dst = edge_index[1]
    mT = jnp.transpose(msg[:, NODE_EMB:NODE_EMB + NR])
    gkey2, skey2 = _prep(mT, src.reshape(1, E), dst.reshape(1, E))
    pad = EP - E
    gkey3 = jnp.reshape(jnp.pad(jnp.reshape(gkey2, (E,)), (0, pad)),
                        (NW, KCH, CH))
    skey3 = jnp.reshape(
        jnp.pad(jnp.reshape(skey2, (E,)), (0, pad), constant_values=NR * N),
        (NW, KCH, CH))
    dst3 = jnp.reshape(jnp.pad(dst, (0, pad), constant_values=N),
                       (NW, KCH, CH))

    cnt2 = _count_kernel(skey3)
    s3 = _scale_kernel(cnt2[:CNT], cnt2[CNT:], skey3)

    ws1 = jnp.concatenate([W1, R1[None]], axis=0)
    ws2 = jnp.concatenate([W2, R2[None]], axis=0)

    y1 = _mm(x, ws1)
    p1 = _agg_kernel(jnp.reshape(y1, ((NR + 1) * N, D)), gkey3, dst3, s3)
    h1 = _combine(p1, y1, jnp.reshape(b1, (1, D)), relu=True)

    y2 = _mm(h1, ws2)
    p2 = _agg_kernel(jnp.reshape(y2, ((NR + 1) * N, D)), gkey3, dst3, s3)
    return _combine(p2, y2, jnp.reshape(b2, (1, D)), relu=False)


def kernel(x, last_update, edge_index, t, msg, W1, R1, b1, W2, R2, b2):
    return _run(x, edge_index, msg, W1, R1, b1, W2, R2, b2)
